# Initial kernel scaffold; baseline (speedup 1.0000x reference)
import jax
import jax.numpy as jnp
from jax.experimental import pallas as pl


def kernel(batch_size, semantic_scores, pt_offsets, batch_idxs, coords_float):
    raise NotImplementedError("write your pallas kernel here")


# speedup vs baseline: 8.1927x; 8.1927x over previous
import functools

import jax
import jax.numpy as jnp
from jax import lax
from jax.experimental import pallas as pl
from jax.experimental.pallas import tpu as pltpu
from jax.experimental.pallas import tpu_sc as plsc

N = 200000
C = 20
LABEL_SHIFT = 2
NUM_BUCKETS = 8192
RADIUS = 0.04
SCORE_THR = 0.05
NCLS = C - LABEL_SHIFT
NC = 2
NS = 16
CLS_PER_CORE = NCLS // NC
CORE_ROWS = CLS_PER_CORE * NUM_BUCKETS

NPAD = 204800
PTS_PER_TILE = NPAD // NS
BLK_PTS = 128
N_BLKS = PTS_PER_TILE // BLK_PTS

ACC_ROWS = 75776
DUMP_ROW = CORE_ROWS
ZERO_CHUNKS = ACC_ROWS // (NS * BLK_PTS)
DRAIN_ROWS = CORE_ROWS // NS

A_BLK = 2560
A_GRID = NPAD // A_BLK

V_W = 24


def _prep_body(scores_ref, off_ref, coords_ref, batch_ref, v_ref, seg_ref, mask_ref):
    scores = scores_ref[...]
    m = jnp.max(scores, axis=1, keepdims=True)
    e = jnp.exp(scores - m)
    p = e / jnp.sum(e, axis=1, keepdims=True)
    keep = (p[:, LABEL_SHIFT:] > SCORE_THR).astype(jnp.int32)
    bits = lax.broadcasted_iota(jnp.int32, (1, NCLS), 1)
    mask_ref[...] = jnp.sum(keep << bits, axis=1, keepdims=True)

    shifted = coords_ref[...] + off_ref[...]
    vox = jnp.floor(shifted / jnp.float32(RADIUS)).astype(jnp.int32)
    h = ((vox[:, 0:1] * 73856093) ^ (vox[:, 1:2] * 19349663)
         ^ (vox[:, 2:3] * 83492791) ^ (batch_ref[...] * 1000003))
    seg_ref[...] = jnp.abs(h) & (NUM_BUCKETS - 1)

    v_ref[...] = jnp.concatenate(
        [scores, shifted, jnp.ones((A_BLK, 1), jnp.float32)], axis=1)


def _prep(scores, offsets, coords, batch):
    return pl.pallas_call(
        _prep_body,
        grid=(A_GRID,),
        in_specs=[
            pl.BlockSpec((A_BLK, C), lambda i: (i, 0)),
            pl.BlockSpec((A_BLK, 3), lambda i: (i, 0)),
            pl.BlockSpec((A_BLK, 3), lambda i: (i, 0)),
            pl.BlockSpec((A_BLK, 1), lambda i: (i, 0)),
        ],
        out_specs=[
            pl.BlockSpec((A_BLK, V_W), lambda i: (i, 0)),
            pl.BlockSpec((A_BLK, 1), lambda i: (i, 0)),
            pl.BlockSpec((A_BLK, 1), lambda i: (i, 0)),
        ],
        out_shape=[
            jax.ShapeDtypeStruct((NPAD, V_W), jnp.float32),
            jax.ShapeDtypeStruct((NPAD, 1), jnp.int32),
            jax.ShapeDtypeStruct((NPAD, 1), jnp.int32),
        ],
    )(scores, offsets, coords, batch)


def _scatter_body(v_hbm, seg_hbm, mask_hbm, out_hbm,
                  v_buf, seg_buf, mask_buf, idx_buf, acc):
    core = lax.axis_index("c")
    sid = lax.axis_index("s")

    zeros16 = jnp.zeros((16,), jnp.float32)
    for r in range(BLK_PTS):
        v_buf[r, pl.ds(0, 16)] = zeros16
        v_buf[r, pl.ds(V_W - 16, 16)] = zeros16

    def zero_step(i, _):
        pltpu.sync_copy(v_buf, acc.at[pl.ds((sid * ZERO_CHUNKS + i) * BLK_PTS,
                                            BLK_PTS)])
        return 0
    lax.fori_loop(0, ZERO_CHUNKS, zero_step, 0)
    plsc.subcore_barrier()

    base = sid * PTS_PER_TILE
    cls_base = core * CLS_PER_CORE

    def block_step(i, _):
        off = base + i * BLK_PTS
        pltpu.sync_copy(v_hbm.at[pl.ds(off, BLK_PTS)], v_buf)
        pltpu.sync_copy(seg_hbm.at[pl.ds(off, BLK_PTS)], seg_buf)
        pltpu.sync_copy(mask_hbm.at[pl.ds(off, BLK_PTS)], mask_buf)
        for c in range(CLS_PER_CORE):
            for j in range(BLK_PTS // 16):
                sv = seg_buf[pl.ds(j * 16, 16)]
                mv = mask_buf[pl.ds(j * 16, 16)]
                kept = (lax.shift_right_logical(mv, cls_base + c) & 1) == 1
                idx_buf[pl.ds(j * 16, 16)] = jnp.where(
                    kept, sv + c * NUM_BUCKETS, DUMP_ROW)
            pltpu.sync_copy(v_buf, acc.at[idx_buf], add=True)
        return 0
    lax.fori_loop(0, N_BLKS, block_step, 0)
    plsc.subcore_barrier()

    pltpu.sync_copy(acc.at[pl.ds(sid * DRAIN_ROWS, DRAIN_ROWS)],
                    out_hbm.at[pl.ds(core * CORE_ROWS + sid * DRAIN_ROWS,
                                     DRAIN_ROWS)])


def _scatter(v, seg, mask):
    f = pl.kernel(
        _scatter_body,
        out_type=jax.ShapeDtypeStruct((NCLS * NUM_BUCKETS, V_W), jnp.float32),
        mesh=plsc.VectorSubcoreMesh(core_axis_name="c", subcore_axis_name="s"),
        compiler_params=pltpu.CompilerParams(use_tc_tiling_on_sc=False),
        scratch_types=[
            pltpu.VMEM((BLK_PTS, V_W), jnp.float32),
            pltpu.VMEM((BLK_PTS,), jnp.int32),
            pltpu.VMEM((BLK_PTS,), jnp.int32),
            pltpu.VMEM((BLK_PTS,), jnp.int32),
            pltpu.VMEM_SHARED((ACC_ROWS, V_W), jnp.float32),
        ],
    )
    return f(v, seg, mask)


def _finalize_body(acc_ref, out_ref):
    acc = acc_ref[...]
    denom = jnp.maximum(acc[:, C + 3:C + 4], 1.0)
    out_ref[...] = acc[:, LABEL_SHIFT:C + 3] / denom


def _finalize(acc):
    rows = NCLS * NUM_BUCKETS
    blk = 4096
    return pl.pallas_call(
        _finalize_body,
        grid=(rows // blk,),
        in_specs=[pl.BlockSpec((blk, V_W), lambda i: (i, 0))],
        out_specs=pl.BlockSpec((blk, 21), lambda i: (i, 0)),
        out_shape=jax.ShapeDtypeStruct((rows, 21), jnp.float32),
    )(acc)


def kernel(batch_size, semantic_scores, pt_offsets, batch_idxs, coords_float):
    pad = NPAD - N
    scores = jnp.pad(semantic_scores, ((0, pad), (0, 0)))
    offsets = jnp.pad(pt_offsets, ((0, pad), (0, 0)))
    coords = jnp.pad(coords_float, ((0, pad), (0, 0)))
    batch = jnp.pad(batch_idxs, (0, pad)).reshape(NPAD, 1)

    v, seg, mask = _prep(scores, offsets, coords, batch)
    acc = _scatter(v, seg.reshape(NPAD), mask.reshape(NPAD))
    return _finalize(acc)


# speedup vs baseline: 18.8864x; 2.3053x over previous
import jax
import jax.numpy as jnp
from jax import lax
from jax.experimental import pallas as pl
from jax.experimental.pallas import tpu as pltpu
from jax.experimental.pallas import tpu_sc as plsc

N = 200000
C = 20
LABEL_SHIFT = 2
NUM_BUCKETS = 8192
RADIUS = 0.04
SCORE_THR = 0.05
NCLS = C - LABEL_SHIFT
NC = 2
NS = 16
CLS_PER_CORE = NCLS // NC
CORE_ROWS = CLS_PER_CORE * NUM_BUCKETS

NPAD = 204800
PTS_PER_TILE = NPAD // NS
BLK_PTS = 128
N_STEPS = PTS_PER_TILE // (2 * BLK_PTS)

ACC_ROWS = CORE_ROWS
ZROWS = ACC_ROWS // NS

A_BLK = 2560
A_GRID = NPAD // A_BLK

V_W = 24


def _prep_body(scores_ref, off_ref, coords_ref, batch_ref, v_ref, sm_ref):
    scores = scores_ref[...]
    m = jnp.max(scores, axis=1, keepdims=True)
    e = jnp.exp(scores - m)
    p = e / jnp.sum(e, axis=1, keepdims=True)
    keep = (p[:, LABEL_SHIFT:] > SCORE_THR).astype(jnp.int32)
    bits = lax.broadcasted_iota(jnp.int32, (1, NCLS), 1) + 13
    mask = jnp.sum(keep << bits, axis=1, keepdims=True)

    shifted = coords_ref[...] + off_ref[...]
    vox = jnp.floor(shifted / jnp.float32(RADIUS)).astype(jnp.int32)
    h = ((vox[:, 0:1] * 73856093) ^ (vox[:, 1:2] * 19349663)
         ^ (vox[:, 2:3] * 83492791) ^ (batch_ref[...] * 1000003))
    sm_ref[...] = (jnp.abs(h) & (NUM_BUCKETS - 1)) | mask

    v_ref[...] = jnp.concatenate(
        [scores, shifted, jnp.ones((A_BLK, 1), jnp.float32)], axis=1)


def _prep(scores, offsets, coords, batch):
    return pl.pallas_call(
        _prep_body,
        grid=(A_GRID,),
        in_specs=[
            pl.BlockSpec((A_BLK, C), lambda i: (i, 0)),
            pl.BlockSpec((A_BLK, 3), lambda i: (i, 0)),
            pl.BlockSpec((A_BLK, 3), lambda i: (i, 0)),
            pl.BlockSpec((A_BLK, 1), lambda i: (i, 0)),
        ],
        out_specs=[
            pl.BlockSpec((A_BLK, V_W), lambda i: (i, 0)),
            pl.BlockSpec((A_BLK, 1), lambda i: (i, 0)),
        ],
        out_shape=[
            jax.ShapeDtypeStruct((NPAD, V_W), jnp.float32),
            jax.ShapeDtypeStruct((NPAD, 1), jnp.int32),
        ],
    )(scores, offsets, coords, batch)


def _scatter_body(v_hbm, sm_hbm, zero_hbm, out_hbm,
                  va, vb, sma, smb, idx2d,
                  sem_in_a, sem_in_b, sem_sc_a, sem_sc_b, acc):
    core = lax.axis_index("c")
    sid = lax.axis_index("s")
    cls_base = core * CLS_PER_CORE

    pltpu.sync_copy(zero_hbm, acc.at[pl.ds(sid * ZROWS, ZROWS)])
    plsc.subcore_barrier()

    base = sid * PTS_PER_TILE
    bufs = ((va, sma, sem_in_a, sem_sc_a, 0), (vb, smb, sem_in_b, sem_sc_b, 9))

    def load(i, vv, ss, sem):
        off = base + i * BLK_PTS
        pltpu.async_copy(v_hbm.at[pl.ds(off, BLK_PTS)], vv, sem)
        pltpu.async_copy(sm_hbm.at[pl.ds(off, BLK_PTS)], ss, sem)

    def wait_load(vv, ss, sem):
        pltpu.make_async_copy(v_hbm.at[pl.ds(0, BLK_PTS)], vv, sem).wait()
        pltpu.make_async_copy(sm_hbm.at[pl.ds(0, BLK_PTS)], ss, sem).wait()

    def fire_scatters(vv, ss, sem, row0):
        segs, msks = [], []
        for j in range(BLK_PTS // 16):
            w = ss[pl.ds(j * 16, 16)]
            segs.append(w & (NUM_BUCKETS - 1))
            msks.append(lax.shift_right_logical(w, 13 + cls_base))
        for c in range(CLS_PER_CORE):
            for j in range(BLK_PTS // 16):
                kept = (lax.shift_right_logical(msks[j], c) & 1) == 1
                idx2d[row0 + c, pl.ds(j * 16, 16)] = jnp.where(
                    kept, segs[j] + c * NUM_BUCKETS, -1)
            pltpu.async_copy(
                vv, acc.at[plsc.Indices(idx2d.at[row0 + c], ignored_value=-1)],
                sem, add=True)

    def drain_scatters(vv, sem, row0):
        for c in range(CLS_PER_CORE):
            pltpu.make_async_copy(
                vv, acc.at[plsc.Indices(idx2d.at[row0 + c], ignored_value=-1)],
                sem).wait()

    load(0, va, sma, sem_in_a)
    load(1, vb, smb, sem_in_b)

    def step(s, _):
        for k, (vv, ss, sem_in, sem_sc, row0) in enumerate(bufs):
            wait_load(vv, ss, sem_in)
            fire_scatters(vv, ss, sem_sc, row0)

        @pl.when(s < N_STEPS - 1)
        def _():
            for k, (vv, ss, sem_in, sem_sc, row0) in enumerate(bufs):
                drain_scatters(vv, sem_sc, row0)
                load(2 * s + 2 + k, vv, ss, sem_in)
        return 0

    lax.fori_loop(0, N_STEPS, step, 0)
    for vv, ss, sem_in, sem_sc, row0 in bufs:
        drain_scatters(vv, sem_sc, row0)
    plsc.subcore_barrier()

    drain = CORE_ROWS // NS
    pltpu.sync_copy(acc.at[pl.ds(sid * drain, drain)],
                    out_hbm.at[pl.ds(core * CORE_ROWS + sid * drain, drain)])


def _scatter(v, sm, zrows):
    f = pl.kernel(
        _scatter_body,
        out_type=jax.ShapeDtypeStruct((NCLS * NUM_BUCKETS, V_W), jnp.float32),
        mesh=plsc.VectorSubcoreMesh(core_axis_name="c", subcore_axis_name="s"),
        compiler_params=pltpu.CompilerParams(use_tc_tiling_on_sc=False),
        scratch_types=[
            pltpu.VMEM((BLK_PTS, V_W), jnp.float32),
            pltpu.VMEM((BLK_PTS, V_W), jnp.float32),
            pltpu.VMEM((BLK_PTS,), jnp.int32),
            pltpu.VMEM((BLK_PTS,), jnp.int32),
            pltpu.VMEM((2 * CLS_PER_CORE, BLK_PTS), jnp.int32),
            pltpu.SemaphoreType.DMA,
            pltpu.SemaphoreType.DMA,
            pltpu.SemaphoreType.DMA,
            pltpu.SemaphoreType.DMA,
            pltpu.VMEM_SHARED((ACC_ROWS, V_W), jnp.float32),
        ],
    )
    return f(v, sm, zrows)


def _finalize_body(acc_ref, out_ref):
    acc = acc_ref[...]
    denom = jnp.maximum(acc[:, C + 3:C + 4], 1.0)
    out_ref[...] = acc[:, LABEL_SHIFT:C + 3] / denom


def _finalize(acc):
    rows = NCLS * NUM_BUCKETS
    blk = 4096
    return pl.pallas_call(
        _finalize_body,
        grid=(rows // blk,),
        in_specs=[pl.BlockSpec((blk, V_W), lambda i: (i, 0))],
        out_specs=pl.BlockSpec((blk, 21), lambda i: (i, 0)),
        out_shape=jax.ShapeDtypeStruct((rows, 21), jnp.float32),
    )(acc)


def kernel(batch_size, semantic_scores, pt_offsets, batch_idxs, coords_float):
    pad = NPAD - N
    scores = jnp.pad(semantic_scores, ((0, pad), (0, 0)))
    offsets = jnp.pad(pt_offsets, ((0, pad), (0, 0)))
    coords = jnp.pad(coords_float, ((0, pad), (0, 0)))
    batch = jnp.pad(batch_idxs, (0, pad)).reshape(NPAD, 1)

    v, sm = _prep(scores, offsets, coords, batch)
    zrows = jnp.zeros((ZROWS, V_W), jnp.float32)
    acc = _scatter(v, sm.reshape(NPAD), zrows)
    return _finalize(acc)


# speedup vs baseline: 37.7174x; 1.9971x over previous
import jax
import jax.numpy as jnp
from jax import lax
from jax.experimental import pallas as pl
from jax.experimental.pallas import tpu as pltpu
from jax.experimental.pallas import tpu_sc as plsc

N = 200000
C = 20
LABEL_SHIFT = 2
NUM_BUCKETS = 8192
RADIUS = 0.04
SCORE_THR = 0.05
NCLS = C - LABEL_SHIFT
NC = 2
NS = 16
CLS_PER_CORE = NCLS // NC
CORE_ROWS = CLS_PER_CORE * NUM_BUCKETS

NPAD = 204800
PTS_PER_TILE = NPAD // NS
BLK_PTS = 128
N_STEPS = PTS_PER_TILE // (2 * BLK_PTS)

ACC_ROWS = CORE_ROWS
ZROWS = ACC_ROWS // NS

A_BLK = 2560
A_GRID = NPAD // A_BLK

V_W = 24


def _prep_body(scores_ref, off_ref, coords_ref, batch_ref, v_ref, sm_ref):
    s = scores_ref[...]
    m = jnp.max(s, axis=0, keepdims=True)
    e = jnp.exp(s - m)
    p = e / jnp.sum(e, axis=0, keepdims=True)
    keep = (p[LABEL_SHIFT:, :] > SCORE_THR).astype(jnp.int32)
    bits = lax.broadcasted_iota(jnp.int32, (NCLS, 1), 0) + 13
    mask = jnp.sum(keep << bits, axis=0, keepdims=True)

    shifted = coords_ref[...] + off_ref[...]
    vox = jnp.floor(shifted / jnp.float32(RADIUS)).astype(jnp.int32)
    h = ((vox[0:1, :] * 73856093) ^ (vox[1:2, :] * 19349663)
         ^ (vox[2:3, :] * 83492791) ^ (batch_ref[...] * 1000003))
    sm_ref[...] = (jnp.abs(h) & (NUM_BUCKETS - 1)) | mask

    vt = jnp.concatenate(
        [s, shifted, jnp.ones((1, A_BLK), jnp.float32)], axis=0)
    v_ref[...] = vt.T


def _prep(scores_t, offsets_t, coords_t, batch_t):
    return pl.pallas_call(
        _prep_body,
        grid=(A_GRID,),
        in_specs=[
            pl.BlockSpec((C, A_BLK), lambda i: (0, i)),
            pl.BlockSpec((3, A_BLK), lambda i: (0, i)),
            pl.BlockSpec((3, A_BLK), lambda i: (0, i)),
            pl.BlockSpec((1, A_BLK), lambda i: (0, i)),
        ],
        out_specs=[
            pl.BlockSpec((A_BLK, V_W), lambda i: (i, 0)),
            pl.BlockSpec((1, A_BLK), lambda i: (0, i)),
        ],
        out_shape=[
            jax.ShapeDtypeStruct((NPAD, V_W), jnp.float32),
            jax.ShapeDtypeStruct((1, NPAD), jnp.int32),
        ],
    )(scores_t, offsets_t, coords_t, batch_t)


def _scatter_body(v_hbm, sm_hbm, zero_hbm, out_hbm,
                  va, vb, sma, smb, idx2d,
                  sem_in_a, sem_in_b, sem_sc_a, sem_sc_b, acc):
    core = lax.axis_index("c")
    sid = lax.axis_index("s")
    cls_base = core * CLS_PER_CORE

    pltpu.sync_copy(zero_hbm, acc.at[pl.ds(sid * ZROWS, ZROWS)])
    plsc.subcore_barrier()

    base = sid * PTS_PER_TILE
    bufs = ((va, sma, sem_in_a, sem_sc_a, 0), (vb, smb, sem_in_b, sem_sc_b, 9))

    def load(i, vv, ss, sem):
        off = base + i * BLK_PTS
        pltpu.async_copy(v_hbm.at[pl.ds(off, BLK_PTS)], vv, sem)
        pltpu.async_copy(sm_hbm.at[pl.ds(off, BLK_PTS)], ss, sem)

    def wait_load(vv, ss, sem):
        pltpu.make_async_copy(v_hbm.at[pl.ds(0, BLK_PTS)], vv, sem).wait()
        pltpu.make_async_copy(sm_hbm.at[pl.ds(0, BLK_PTS)], ss, sem).wait()

    def fire_scatters(vv, ss, sem, row0):
        segs, msks = [], []
        for j in range(BLK_PTS // 16):
            w = ss[pl.ds(j * 16, 16)]
            segs.append(w & (NUM_BUCKETS - 1))
            msks.append(lax.shift_right_logical(w, 13 + cls_base))
        for c in range(CLS_PER_CORE):
            for j in range(BLK_PTS // 16):
                kept = (lax.shift_right_logical(msks[j], c) & 1) == 1
                idx2d[row0 + c, pl.ds(j * 16, 16)] = jnp.where(
                    kept, segs[j] + c * NUM_BUCKETS, -1)
            pltpu.async_copy(
                vv, acc.at[plsc.Indices(idx2d.at[row0 + c], ignored_value=-1)],
                sem, add=True)

    def drain_scatters(vv, sem, row0):
        for c in range(CLS_PER_CORE):
            pltpu.make_async_copy(
                vv, acc.at[plsc.Indices(idx2d.at[row0 + c], ignored_value=-1)],
                sem).wait()

    load(0, va, sma, sem_in_a)
    load(1, vb, smb, sem_in_b)

    def step(s, _):
        for k, (vv, ss, sem_in, sem_sc, row0) in enumerate(bufs):
            wait_load(vv, ss, sem_in)
            fire_scatters(vv, ss, sem_sc, row0)

        @pl.when(s < N_STEPS - 1)
        def _():
            for k, (vv, ss, sem_in, sem_sc, row0) in enumerate(bufs):
                drain_scatters(vv, sem_sc, row0)
                load(2 * s + 2 + k, vv, ss, sem_in)
        return 0

    lax.fori_loop(0, N_STEPS, step, 0)
    for vv, ss, sem_in, sem_sc, row0 in bufs:
        drain_scatters(vv, sem_sc, row0)
    plsc.subcore_barrier()

    drain = CORE_ROWS // NS
    pltpu.sync_copy(acc.at[pl.ds(sid * drain, drain)],
                    out_hbm.at[pl.ds(core * CORE_ROWS + sid * drain, drain)])


def _scatter(v, sm, zrows):
    f = pl.kernel(
        _scatter_body,
        out_type=jax.ShapeDtypeStruct((NCLS * NUM_BUCKETS, V_W), jnp.float32),
        mesh=plsc.VectorSubcoreMesh(core_axis_name="c", subcore_axis_name="s"),
        compiler_params=pltpu.CompilerParams(use_tc_tiling_on_sc=False),
        scratch_types=[
            pltpu.VMEM((BLK_PTS, V_W), jnp.float32),
            pltpu.VMEM((BLK_PTS, V_W), jnp.float32),
            pltpu.VMEM((BLK_PTS,), jnp.int32),
            pltpu.VMEM((BLK_PTS,), jnp.int32),
            pltpu.VMEM((2 * CLS_PER_CORE, BLK_PTS), jnp.int32),
            pltpu.SemaphoreType.DMA,
            pltpu.SemaphoreType.DMA,
            pltpu.SemaphoreType.DMA,
            pltpu.SemaphoreType.DMA,
            pltpu.VMEM_SHARED((ACC_ROWS, V_W), jnp.float32),
        ],
    )
    return f(v, sm, zrows)


def _finalize_body(acc_ref, out_ref):
    acc = acc_ref[...]
    denom = jnp.maximum(acc[:, C + 3:C + 4], 1.0)
    out_ref[...] = acc[:, LABEL_SHIFT:C + 3] / denom


def _finalize(acc):
    rows = NCLS * NUM_BUCKETS
    blk = 4096
    return pl.pallas_call(
        _finalize_body,
        grid=(rows // blk,),
        in_specs=[pl.BlockSpec((blk, V_W), lambda i: (i, 0))],
        out_specs=pl.BlockSpec((blk, 21), lambda i: (i, 0)),
        out_shape=jax.ShapeDtypeStruct((rows, 21), jnp.float32),
    )(acc)


def kernel(batch_size, semantic_scores, pt_offsets, batch_idxs, coords_float):
    pad = NPAD - N
    scores_t = jnp.pad(semantic_scores.T, ((0, 0), (0, pad)))
    offsets_t = jnp.pad(pt_offsets.T, ((0, 0), (0, pad)))
    coords_t = jnp.pad(coords_float.T, ((0, 0), (0, pad)))
    batch_t = jnp.pad(batch_idxs, (0, pad)).reshape(1, NPAD)

    v, sm = _prep(scores_t, offsets_t, coords_t, batch_t)
    zrows = jnp.zeros((ZROWS, V_W), jnp.float32)
    acc = _scatter(v, sm.reshape(NPAD), zrows)
    return _finalize(acc)


# speedup vs baseline: 40.0928x; 1.0630x over previous
import jax
import jax.numpy as jnp
from jax import lax
from jax.experimental import pallas as pl
from jax.experimental.pallas import tpu as pltpu
from jax.experimental.pallas import tpu_sc as plsc

N = 200000
C = 20
LABEL_SHIFT = 2
NUM_BUCKETS = 8192
RADIUS = 0.04
SCORE_THR = 0.05
NCLS = C - LABEL_SHIFT
NC = 2
NS = 16
CLS_PER_CORE = NCLS // NC
CORE_ROWS = CLS_PER_CORE * NUM_BUCKETS

NPAD = 204800
PTS_PER_TILE = NPAD // NS
BLK_PTS = 128
N_STEPS = PTS_PER_TILE // (2 * BLK_PTS)

ACC_ROWS = CORE_ROWS
ZROWS = ACC_ROWS // NS

A_BLK = 2560
A_GRID = NPAD // A_BLK

V_W = 24


def _prep_body(scores_ref, off_ref, coords_ref, batch_ref, v_ref, sm_ref):
    s = scores_ref[...]
    m = jnp.max(s, axis=0, keepdims=True)
    e = jnp.exp(s - m)
    p = e / jnp.sum(e, axis=0, keepdims=True)
    keep = (p[LABEL_SHIFT:, :] > SCORE_THR).astype(jnp.int32)
    bits = lax.broadcasted_iota(jnp.int32, (NCLS, 1), 0) + 13
    mask = jnp.sum(keep << bits, axis=0, keepdims=True)

    shifted = coords_ref[...] + off_ref[...]
    vox = jnp.floor(shifted / jnp.float32(RADIUS)).astype(jnp.int32)
    h = ((vox[0:1, :] * 73856093) ^ (vox[1:2, :] * 19349663)
         ^ (vox[2:3, :] * 83492791) ^ (batch_ref[...] * 1000003))
    sm_ref[...] = (jnp.abs(h) & (NUM_BUCKETS - 1)) | mask

    vt = jnp.concatenate(
        [s, shifted, jnp.ones((1, A_BLK), jnp.float32)], axis=0)
    v_ref[...] = vt.T


def _prep(scores_t, offsets_t, coords_t, batch_t):
    return pl.pallas_call(
        _prep_body,
        grid=(A_GRID,),
        in_specs=[
            pl.BlockSpec((C, A_BLK), lambda i: (0, i)),
            pl.BlockSpec((3, A_BLK), lambda i: (0, i)),
            pl.BlockSpec((3, A_BLK), lambda i: (0, i)),
            pl.BlockSpec((1, A_BLK), lambda i: (0, i)),
        ],
        out_specs=[
            pl.BlockSpec((A_BLK, V_W), lambda i: (i, 0)),
            pl.BlockSpec((1, A_BLK), lambda i: (0, i)),
        ],
        out_shape=[
            jax.ShapeDtypeStruct((NPAD, V_W), jnp.float32),
            jax.ShapeDtypeStruct((1, NPAD), jnp.int32),
        ],
    )(scores_t, offsets_t, coords_t, batch_t)


def _scatter_body(v_hbm, sm_hbm, zero_hbm, out_hbm,
                  va, vb, sma, smb, idx2d, fin_out,
                  sem_in_a, sem_in_b, sem_sc_a, sem_sc_b, acc):
    core = lax.axis_index("c")
    sid = lax.axis_index("s")
    cls_base = core * CLS_PER_CORE

    pltpu.sync_copy(zero_hbm, acc.at[pl.ds(sid * ZROWS, ZROWS)])
    plsc.subcore_barrier()

    base = sid * PTS_PER_TILE
    bufs = ((va, sma, sem_in_a, sem_sc_a, 0), (vb, smb, sem_in_b, sem_sc_b, 9))

    def load(i, vv, ss, sem):
        off = base + i * BLK_PTS
        pltpu.async_copy(v_hbm.at[pl.ds(off, BLK_PTS)], vv, sem)
        pltpu.async_copy(sm_hbm.at[pl.ds(off, BLK_PTS)], ss, sem)

    def wait_load(vv, ss, sem):
        pltpu.make_async_copy(v_hbm.at[pl.ds(0, BLK_PTS)], vv, sem).wait()
        pltpu.make_async_copy(sm_hbm.at[pl.ds(0, BLK_PTS)], ss, sem).wait()

    def fire_scatters(vv, ss, sem, row0):
        segs, msks = [], []
        for j in range(BLK_PTS // 16):
            w = ss[pl.ds(j * 16, 16)]
            segs.append(w & (NUM_BUCKETS - 1))
            msks.append(lax.shift_right_logical(w, 13 + cls_base))
        for c in range(CLS_PER_CORE):
            for j in range(BLK_PTS // 16):
                kept = (lax.shift_right_logical(msks[j], c) & 1) == 1
                idx2d[row0 + c, pl.ds(j * 16, 16)] = jnp.where(
                    kept, segs[j] + c * NUM_BUCKETS, -1)
            pltpu.async_copy(
                vv, acc.at[plsc.Indices(idx2d.at[row0 + c], ignored_value=-1)],
                sem, add=True)

    def drain_scatters(vv, sem, row0):
        for c in range(CLS_PER_CORE):
            pltpu.make_async_copy(
                vv, acc.at[plsc.Indices(idx2d.at[row0 + c], ignored_value=-1)],
                sem).wait()

    load(0, va, sma, sem_in_a)
    load(1, vb, smb, sem_in_b)

    def step(s, _):
        for k, (vv, ss, sem_in, sem_sc, row0) in enumerate(bufs):
            wait_load(vv, ss, sem_in)
            fire_scatters(vv, ss, sem_sc, row0)

        @pl.when(s < N_STEPS - 1)
        def _():
            for k, (vv, ss, sem_in, sem_sc, row0) in enumerate(bufs):
                drain_scatters(vv, sem_sc, row0)
                load(2 * s + 2 + k, vv, ss, sem_in)
        return 0

    lax.fori_loop(0, N_STEPS, step, 0)
    for vv, ss, sem_in, sem_sc, row0 in bufs:
        drain_scatters(vv, sem_sc, row0)
    plsc.subcore_barrier()

    drain = CORE_ROWS // NS
    FCH = BLK_PTS
    fin_in = va

    def fin_chunk(k, _):
        pltpu.sync_copy(acc.at[pl.ds(sid * drain + k * FCH, FCH)], fin_in)

        def fin_row(r, _):
            hi = fin_in[r, pl.ds(7, 16)]
            cntv = fin_in[r, pl.ds(8, 16)]
            inv = (1.0 / jnp.maximum(cntv, 1.0))[15]
            fin_out[r, pl.ds(0, 16)] = fin_in[r, pl.ds(2, 16)] * inv
            fin_out[r, pl.ds(5, 16)] = hi * inv
            return 0
        lax.fori_loop(0, FCH, fin_row, 0)
        pltpu.sync_copy(
            fin_out,
            out_hbm.at[pl.ds(core * CORE_ROWS + sid * drain + k * FCH, FCH)])
        return 0
    lax.fori_loop(0, drain // FCH, fin_chunk, 0)

def _scatter(v, sm, zrows):
    f = pl.kernel(
        _scatter_body,
        out_type=jax.ShapeDtypeStruct((NCLS * NUM_BUCKETS, 21), jnp.float32),
        mesh=plsc.VectorSubcoreMesh(core_axis_name="c", subcore_axis_name="s"),
        compiler_params=pltpu.CompilerParams(use_tc_tiling_on_sc=False),
        scratch_types=[
            pltpu.VMEM((BLK_PTS, V_W), jnp.float32),
            pltpu.VMEM((BLK_PTS, V_W), jnp.float32),
            pltpu.VMEM((BLK_PTS,), jnp.int32),
            pltpu.VMEM((BLK_PTS,), jnp.int32),
            pltpu.VMEM((2 * CLS_PER_CORE, BLK_PTS), jnp.int32),
            pltpu.VMEM((BLK_PTS, 21), jnp.float32),
            pltpu.SemaphoreType.DMA,
            pltpu.SemaphoreType.DMA,
            pltpu.SemaphoreType.DMA,
            pltpu.SemaphoreType.DMA,
            pltpu.VMEM_SHARED((ACC_ROWS, V_W), jnp.float32),
        ],
    )
    return f(v, sm, zrows)


def kernel(batch_size, semantic_scores, pt_offsets, batch_idxs, coords_float):
    pad = NPAD - N
    scores_t = jnp.pad(semantic_scores.T, ((0, 0), (0, pad)))
    offsets_t = jnp.pad(pt_offsets.T, ((0, 0), (0, pad)))
    coords_t = jnp.pad(coords_float.T, ((0, 0), (0, pad)))
    batch_t = jnp.pad(batch_idxs, (0, pad)).reshape(1, NPAD)

    v, sm = _prep(scores_t, offsets_t, coords_t, batch_t)
    zrows = jnp.zeros((ZROWS, V_W), jnp.float32)
    return _scatter(v, sm.reshape(NPAD), zrows)


# speedup vs baseline: 41.8384x; 1.0435x over previous
import jax
import jax.numpy as jnp
from jax import lax
from jax.experimental import pallas as pl
from jax.experimental.pallas import tpu as pltpu
from jax.experimental.pallas import tpu_sc as plsc

N = 200000
C = 20
LABEL_SHIFT = 2
NUM_BUCKETS = 8192
RADIUS = 0.04
SCORE_THR = 0.05
NCLS = C - LABEL_SHIFT
NC = 2
NS = 16
CLS_PER_CORE = NCLS // NC
CORE_ROWS = CLS_PER_CORE * NUM_BUCKETS

NPAD = 204800
PTS_PER_TILE = NPAD // NS
BLK_PTS = 128
N_STEPS = PTS_PER_TILE // (2 * BLK_PTS)

ACC_ROWS = CORE_ROWS
ZROWS = ACC_ROWS // NS

A_BLK = 5120
A_GRID = NPAD // A_BLK

V_W = 24


def _prep_body(scores_ref, off_ref, coords_ref, batch_ref, v_ref, sm_ref):
    s = scores_ref[...]
    m = jnp.max(s, axis=0, keepdims=True)
    e = jnp.exp(s - m)
    p = e / jnp.sum(e, axis=0, keepdims=True)
    keep = (p[LABEL_SHIFT:, :] > SCORE_THR).astype(jnp.int32)
    bits = lax.broadcasted_iota(jnp.int32, (NCLS, 1), 0) + 13
    mask = jnp.sum(keep << bits, axis=0, keepdims=True)

    shifted = coords_ref[...] + off_ref[...]
    vox = jnp.floor(shifted / jnp.float32(RADIUS)).astype(jnp.int32)
    h = ((vox[0:1, :] * 73856093) ^ (vox[1:2, :] * 19349663)
         ^ (vox[2:3, :] * 83492791) ^ (batch_ref[...] * 1000003))
    sm_ref[...] = (jnp.abs(h) & (NUM_BUCKETS - 1)) | mask

    vt = jnp.concatenate(
        [s, shifted, jnp.ones((1, A_BLK), jnp.float32)], axis=0)
    v_ref[...] = vt.T


def _prep(scores_t, offsets_t, coords_t, batch_t):
    return pl.pallas_call(
        _prep_body,
        grid=(A_GRID,),
        in_specs=[
            pl.BlockSpec((C, A_BLK), lambda i: (0, i)),
            pl.BlockSpec((3, A_BLK), lambda i: (0, i)),
            pl.BlockSpec((3, A_BLK), lambda i: (0, i)),
            pl.BlockSpec((1, A_BLK), lambda i: (0, i)),
        ],
        out_specs=[
            pl.BlockSpec((A_BLK, V_W), lambda i: (i, 0)),
            pl.BlockSpec((1, A_BLK), lambda i: (0, i)),
        ],
        out_shape=[
            jax.ShapeDtypeStruct((NPAD, V_W), jnp.float32),
            jax.ShapeDtypeStruct((1, NPAD), jnp.int32),
        ],
    )(scores_t, offsets_t, coords_t, batch_t)


def _scatter_body(v_hbm, sm_hbm, zero_hbm, out_hbm,
                  va, vb, sma, smb, idx2d, fin_out,
                  sem_in_a, sem_in_b, sem_sc_a, sem_sc_b, acc):
    core = lax.axis_index("c")
    sid = lax.axis_index("s")
    cls_base = core * CLS_PER_CORE

    pltpu.sync_copy(zero_hbm, acc.at[pl.ds(sid * ZROWS, ZROWS)])
    plsc.subcore_barrier()

    base = sid * PTS_PER_TILE
    bufs = ((va, sma, sem_in_a, sem_sc_a, 0), (vb, smb, sem_in_b, sem_sc_b, 9))

    def load(i, vv, ss, sem):
        off = base + i * BLK_PTS
        pltpu.async_copy(v_hbm.at[pl.ds(off, BLK_PTS)], vv, sem)
        pltpu.async_copy(sm_hbm.at[pl.ds(off, BLK_PTS)], ss, sem)

    def wait_load(vv, ss, sem):
        pltpu.make_async_copy(v_hbm.at[pl.ds(0, BLK_PTS)], vv, sem).wait()
        pltpu.make_async_copy(sm_hbm.at[pl.ds(0, BLK_PTS)], ss, sem).wait()

    def fire_scatters(vv, ss, sem, row0):
        segs, msks = [], []
        for j in range(BLK_PTS // 16):
            w = ss[pl.ds(j * 16, 16)]
            segs.append(w & (NUM_BUCKETS - 1))
            msks.append(lax.shift_right_logical(w, 13 + cls_base))
        for c in range(CLS_PER_CORE):
            for j in range(BLK_PTS // 16):
                kept = (lax.shift_right_logical(msks[j], c) & 1) == 1
                idx2d[row0 + c, pl.ds(j * 16, 16)] = jnp.where(
                    kept, segs[j] + c * NUM_BUCKETS, -1)
            pltpu.async_copy(
                vv, acc.at[plsc.Indices(idx2d.at[row0 + c], ignored_value=-1)],
                sem, add=True)

    def drain_scatters(vv, sem, row0):
        for c in range(CLS_PER_CORE):
            pltpu.make_async_copy(
                vv, acc.at[plsc.Indices(idx2d.at[row0 + c], ignored_value=-1)],
                sem).wait()

    load(0, va, sma, sem_in_a)
    load(1, vb, smb, sem_in_b)

    def step(s, _):
        for k, (vv, ss, sem_in, sem_sc, row0) in enumerate(bufs):
            wait_load(vv, ss, sem_in)
            fire_scatters(vv, ss, sem_sc, row0)

        @pl.when(s < N_STEPS - 1)
        def _():
            for k, (vv, ss, sem_in, sem_sc, row0) in enumerate(bufs):
                drain_scatters(vv, sem_sc, row0)
                load(2 * s + 2 + k, vv, ss, sem_in)
        return 0

    lax.fori_loop(0, N_STEPS, step, 0)
    for vv, ss, sem_in, sem_sc, row0 in bufs:
        drain_scatters(vv, sem_sc, row0)
    plsc.subcore_barrier()

    drain = CORE_ROWS // NS
    FCH = BLK_PTS
    fin_in = va

    def fin_chunk(k, _):
        pltpu.sync_copy(acc.at[pl.ds(sid * drain + k * FCH, FCH)], fin_in)

        def fin_row(r, _):
            hi = fin_in[r, pl.ds(7, 16)]
            cntv = fin_in[r, pl.ds(8, 16)]
            inv = (1.0 / jnp.maximum(cntv, 1.0))[15]
            fin_out[r, pl.ds(0, 16)] = fin_in[r, pl.ds(2, 16)] * inv
            fin_out[r, pl.ds(5, 16)] = hi * inv
            return 0
        lax.fori_loop(0, FCH, fin_row, 0)
        pltpu.sync_copy(
            fin_out,
            out_hbm.at[pl.ds(core * CORE_ROWS + sid * drain + k * FCH, FCH)])
        return 0
    lax.fori_loop(0, drain // FCH, fin_chunk, 0)

def _scatter(v, sm, zrows):
    f = pl.kernel(
        _scatter_body,
        out_type=jax.ShapeDtypeStruct((NCLS * NUM_BUCKETS, 21), jnp.float32),
        mesh=plsc.VectorSubcoreMesh(core_axis_name="c", subcore_axis_name="s"),
        compiler_params=pltpu.CompilerParams(use_tc_tiling_on_sc=False),
        scratch_types=[
            pltpu.VMEM((BLK_PTS, V_W), jnp.float32),
            pltpu.VMEM((BLK_PTS, V_W), jnp.float32),
            pltpu.VMEM((BLK_PTS,), jnp.int32),
            pltpu.VMEM((BLK_PTS,), jnp.int32),
            pltpu.VMEM((2 * CLS_PER_CORE, BLK_PTS), jnp.int32),
            pltpu.VMEM((BLK_PTS, 21), jnp.float32),
            pltpu.SemaphoreType.DMA,
            pltpu.SemaphoreType.DMA,
            pltpu.SemaphoreType.DMA,
            pltpu.SemaphoreType.DMA,
            pltpu.VMEM_SHARED((ACC_ROWS, V_W), jnp.float32),
        ],
    )
    return f(v, sm, zrows)


def kernel(batch_size, semantic_scores, pt_offsets, batch_idxs, coords_float):
    pad = NPAD - N
    scores_t = jnp.pad(semantic_scores.T, ((0, 0), (0, pad)))
    offsets_t = jnp.pad(pt_offsets.T, ((0, 0), (0, pad)))
    coords_t = jnp.pad(coords_float.T, ((0, 0), (0, pad)))
    batch_t = jnp.pad(batch_idxs, (0, pad)).reshape(1, NPAD)

    v, sm = _prep(scores_t, offsets_t, coords_t, batch_t)
    zrows = jnp.zeros((ZROWS, V_W), jnp.float32)
    return _scatter(v, sm.reshape(NPAD), zrows)


# speedup vs baseline: 41.9441x; 1.0025x over previous
import jax
import jax.numpy as jnp
from jax import lax
from jax.experimental import pallas as pl
from jax.experimental.pallas import tpu as pltpu
from jax.experimental.pallas import tpu_sc as plsc

N = 200000
C = 20
LABEL_SHIFT = 2
NUM_BUCKETS = 8192
RADIUS = 0.04
SCORE_THR = 0.05
NCLS = C - LABEL_SHIFT
NC = 2
NS = 16
CLS_PER_CORE = NCLS // NC
CORE_ROWS = CLS_PER_CORE * NUM_BUCKETS

NPAD = 204800
PTS_PER_TILE = NPAD // NS
BLK_PTS = 128
N_STEPS = PTS_PER_TILE // (2 * BLK_PTS)

ACC_ROWS = CORE_ROWS
ZROWS = ACC_ROWS // NS

A_BLK = 5120
A_GRID = NPAD // A_BLK

V_W = 24


def _prep_body(scores_ref, off_ref, coords_ref, batch_ref, v_ref, sm_ref):
    s = scores_ref[...]
    m = jnp.max(s, axis=0, keepdims=True)
    e = jnp.exp(s - m)
    p = e / jnp.sum(e, axis=0, keepdims=True)
    keep = (p[LABEL_SHIFT:, :] > SCORE_THR).astype(jnp.int32)
    bits = lax.broadcasted_iota(jnp.int32, (NCLS, 1), 0) + 13
    mask = jnp.sum(keep << bits, axis=0, keepdims=True)

    shifted = coords_ref[...] + off_ref[...]
    vox = jnp.floor(shifted / jnp.float32(RADIUS)).astype(jnp.int32)
    h = ((vox[0:1, :] * 73856093) ^ (vox[1:2, :] * 19349663)
         ^ (vox[2:3, :] * 83492791) ^ (batch_ref[...] * 1000003))
    sm_ref[...] = ((jnp.abs(h) & (NUM_BUCKETS - 1)) | mask).reshape(A_BLK)

    vt = jnp.concatenate(
        [s, shifted, jnp.ones((1, A_BLK), jnp.float32)], axis=0)
    v_ref[...] = vt.T


def _prep(scores_t, offsets_t, coords_t, batch_t):
    return pl.pallas_call(
        _prep_body,
        grid=(A_GRID,),
        in_specs=[
            pl.BlockSpec((C, A_BLK), lambda i: (0, i)),
            pl.BlockSpec((3, A_BLK), lambda i: (0, i)),
            pl.BlockSpec((3, A_BLK), lambda i: (0, i)),
            pl.BlockSpec((1, A_BLK), lambda i: (0, i)),
        ],
        out_specs=[
            pl.BlockSpec((A_BLK, V_W), lambda i: (i, 0)),
            pl.BlockSpec((A_BLK,), lambda i: (i,)),
        ],
        out_shape=[
            jax.ShapeDtypeStruct((NPAD, V_W), jnp.float32),
            jax.ShapeDtypeStruct((NPAD,), jnp.int32),
        ],
    )(scores_t, offsets_t, coords_t, batch_t)


def _scatter_body(v_hbm, sm_hbm, zero_hbm, out_hbm,
                  va, vb, sma, smb, idx2d, fin_out,
                  sem_in_a, sem_in_b, sem_sc_a, sem_sc_b, acc):
    core = lax.axis_index("c")
    sid = lax.axis_index("s")
    cls_base = core * CLS_PER_CORE

    pltpu.sync_copy(zero_hbm, acc.at[pl.ds(sid * ZROWS, ZROWS)])
    plsc.subcore_barrier()

    base = sid * PTS_PER_TILE
    bufs = ((va, sma, sem_in_a, sem_sc_a, 0), (vb, smb, sem_in_b, sem_sc_b, 9))

    def load(i, vv, ss, sem):
        off = base + i * BLK_PTS
        pltpu.async_copy(v_hbm.at[pl.ds(off, BLK_PTS)], vv, sem)
        pltpu.async_copy(sm_hbm.at[pl.ds(off, BLK_PTS)], ss, sem)

    def wait_load(vv, ss, sem):
        pltpu.make_async_copy(v_hbm.at[pl.ds(0, BLK_PTS)], vv, sem).wait()
        pltpu.make_async_copy(sm_hbm.at[pl.ds(0, BLK_PTS)], ss, sem).wait()

    def fire_scatters(vv, ss, sem, row0):
        segs, msks = [], []
        for j in range(BLK_PTS // 16):
            w = ss[pl.ds(j * 16, 16)]
            segs.append(w & (NUM_BUCKETS - 1))
            msks.append(lax.shift_right_logical(w, 13 + cls_base))
        for c in range(CLS_PER_CORE):
            for j in range(BLK_PTS // 16):
                kept = (lax.shift_right_logical(msks[j], c) & 1) == 1
                idx2d[row0 + c, pl.ds(j * 16, 16)] = jnp.where(
                    kept, segs[j] + c * NUM_BUCKETS, -1)
            pltpu.async_copy(
                vv, acc.at[plsc.Indices(idx2d.at[row0 + c], ignored_value=-1)],
                sem, add=True)

    def drain_scatters(vv, sem, row0):
        for c in range(CLS_PER_CORE):
            pltpu.make_async_copy(
                vv, acc.at[plsc.Indices(idx2d.at[row0 + c], ignored_value=-1)],
                sem).wait()

    load(0, va, sma, sem_in_a)
    load(1, vb, smb, sem_in_b)

    def step(s, _):
        for k, (vv, ss, sem_in, sem_sc, row0) in enumerate(bufs):
            wait_load(vv, ss, sem_in)
            fire_scatters(vv, ss, sem_sc, row0)

        @pl.when(s < N_STEPS - 1)
        def _():
            for k, (vv, ss, sem_in, sem_sc, row0) in enumerate(bufs):
                drain_scatters(vv, sem_sc, row0)
                load(2 * s + 2 + k, vv, ss, sem_in)
        return 0

    lax.fori_loop(0, N_STEPS, step, 0)
    for vv, ss, sem_in, sem_sc, row0 in bufs:
        drain_scatters(vv, sem_sc, row0)
    plsc.subcore_barrier()

    drain = CORE_ROWS // NS
    FCH = BLK_PTS
    fin_in = va

    def fin_chunk(k, _):
        pltpu.sync_copy(acc.at[pl.ds(sid * drain + k * FCH, FCH)], fin_in)

        def fin_row(r, _):
            hi = fin_in[r, pl.ds(7, 16)]
            cntv = fin_in[r, pl.ds(8, 16)]
            inv = (1.0 / jnp.maximum(cntv, 1.0))[15]
            fin_out[r, pl.ds(0, 16)] = fin_in[r, pl.ds(2, 16)] * inv
            fin_out[r, pl.ds(5, 16)] = hi * inv
            return 0
        lax.fori_loop(0, FCH, fin_row, 0)
        pltpu.sync_copy(
            fin_out,
            out_hbm.at[pl.ds(core * CORE_ROWS + sid * drain + k * FCH, FCH)])
        return 0
    lax.fori_loop(0, drain // FCH, fin_chunk, 0)

def _scatter(v, sm, zrows):
    f = pl.kernel(
        _scatter_body,
        out_type=jax.ShapeDtypeStruct((NCLS * NUM_BUCKETS, 21), jnp.float32),
        mesh=plsc.VectorSubcoreMesh(core_axis_name="c", subcore_axis_name="s"),
        compiler_params=pltpu.CompilerParams(use_tc_tiling_on_sc=False),
        scratch_types=[
            pltpu.VMEM((BLK_PTS, V_W), jnp.float32),
            pltpu.VMEM((BLK_PTS, V_W), jnp.float32),
            pltpu.VMEM((BLK_PTS,), jnp.int32),
            pltpu.VMEM((BLK_PTS,), jnp.int32),
            pltpu.VMEM((2 * CLS_PER_CORE, BLK_PTS), jnp.int32),
            pltpu.VMEM((BLK_PTS, 21), jnp.float32),
            pltpu.SemaphoreType.DMA,
            pltpu.SemaphoreType.DMA,
            pltpu.SemaphoreType.DMA,
            pltpu.SemaphoreType.DMA,
            pltpu.VMEM_SHARED((ACC_ROWS, V_W), jnp.float32),
        ],
    )
    return f(v, sm, zrows)


def kernel(batch_size, semantic_scores, pt_offsets, batch_idxs, coords_float):
    pad = NPAD - N
    scores_t = jnp.pad(semantic_scores.T, ((0, 0), (0, pad)))
    offsets_t = jnp.pad(pt_offsets.T, ((0, 0), (0, pad)))
    coords_t = jnp.pad(coords_float.T, ((0, 0), (0, pad)))
    batch_t = jnp.pad(batch_idxs, (0, pad)).reshape(1, NPAD)

    v, sm = _prep(scores_t, offsets_t, coords_t, batch_t)
    zrows = jnp.zeros((ZROWS, V_W), jnp.float32)
    return _scatter(v, sm, zrows)
